# baseline (speedup 1.0000x reference)
import functools

import jax
import jax.numpy as jnp
from jax import lax
from jax.experimental import pallas as pl
from jax.experimental.pallas import tpu as pltpu
from jax.experimental.pallas import tpu_sc as plsc

N = 10000
NP = 10240
E = 160000
NSUB = 16
NCORE = 2
NW = NCORE * NSUB
CH = 128
EP = 163840
NCH = EP // (NW * CH)
STRIPE = NP // NSUB
MBLK = 256
MT = NP // MBLK
NBUF = 4
F_PAD = 1536
H = 60
HP = 64
W3H = 3 * HP


def _fill_rows(ref, nrows, width, value):
    vec = jnp.full((16,), value, jnp.float32)

    def body(i, carry):
        for j in range(width // 16):
            ref[i, pl.ds(j * 16, 16)] = vec
        return carry

    lax.fori_loop(0, nrows, body, 0)


def _make_prop(width, nbuf):
    mesh = plsc.VectorSubcoreMesh(core_axis_name="c", subcore_axis_name="s")

    @functools.partial(
        pl.kernel,
        out_type=jax.ShapeDtypeStruct((NCORE, NP, width), jnp.float32),
        mesh=mesh,
        scratch_types=(
            [pltpu.VMEM((NCH, CH), jnp.int32)] * 2
            + [pltpu.VMEM((CH, width), jnp.float32)] * nbuf
            + [pltpu.VMEM_SHARED((NP, width), jnp.float32)]
            + [pltpu.SemaphoreType.DMA] * (2 * nbuf)
        ),
        compiler_params=pltpu.CompilerParams(use_tc_tiling_on_sc=False),
    )
    def k(table, rowi, coli, out, rowv, colv, *rest):
        gbufs = rest[:nbuf]
        acc = rest[nbuf]
        gsems = rest[nbuf + 1:2 * nbuf + 1]
        ssems = rest[2 * nbuf + 1:3 * nbuf + 1]
        c = lax.axis_index("c")
        s = lax.axis_index("s")
        wid = c * NSUB + s
        _fill_rows(gbufs[0], CH, width, 0.0)
        for r in range(STRIPE // CH):
            pltpu.sync_copy(gbufs[0], acc.at[pl.ds(s * STRIPE + r * CH, CH)])
        pltpu.sync_copy(rowi.at[wid], rowv)
        pltpu.sync_copy(coli.at[wid], colv)
        plsc.subcore_barrier()

        for b in range(nbuf):
            pltpu.async_copy(table.at[rowv.at[b]], gbufs[b], gsems[b])

        def body(i, carry):
            for b in range(nbuf):
                j = nbuf * i + b
                pltpu.make_async_copy(
                    table.at[rowv.at[j]], gbufs[b], gsems[b]).wait()
                pltpu.async_copy(gbufs[b], acc.at[colv.at[j]], ssems[b],
                                 add=True)
            for b in range(nbuf):
                j = nbuf * i + b
                pltpu.make_async_copy(
                    gbufs[b], acc.at[colv.at[j]], ssems[b]).wait()

                @pl.when(j + nbuf < NCH)
                def _():
                    pltpu.async_copy(table.at[rowv.at[j + nbuf]], gbufs[b],
                                     gsems[b])
            return carry

        lax.fori_loop(0, NCH // nbuf, body, 0)
        plsc.subcore_barrier()
        pltpu.sync_copy(acc.at[pl.ds(s * STRIPE, STRIPE)],
                        out.at[c, pl.ds(s * STRIPE, STRIPE)])

    return k


def _make_deg():
    width = 16
    mesh = plsc.VectorSubcoreMesh(core_axis_name="c", subcore_axis_name="s")

    @functools.partial(
        pl.kernel,
        out_type=jax.ShapeDtypeStruct((NCORE, NP, width), jnp.float32),
        mesh=mesh,
        scratch_types=[
            pltpu.VMEM((NCH, CH), jnp.int32),
            pltpu.VMEM((CH, width), jnp.float32),
            pltpu.VMEM((CH, width), jnp.float32),
            pltpu.VMEM_SHARED((NP, width), jnp.float32),
            pltpu.SemaphoreType.DMA,
        ],
        compiler_params=pltpu.CompilerParams(use_tc_tiling_on_sc=False),
    )
    def k(coli, out, colv, obuf, zbuf, acc, sem):
        c = lax.axis_index("c")
        s = lax.axis_index("s")
        wid = c * NSUB + s
        _fill_rows(zbuf, CH, width, 0.0)
        _fill_rows(obuf, CH, width, 1.0)
        for r in range(STRIPE // CH):
            pltpu.sync_copy(zbuf, acc.at[pl.ds(s * STRIPE + r * CH, CH)])
        pltpu.sync_copy(coli.at[wid], colv)
        plsc.subcore_barrier()

        for b in range(NBUF):
            pltpu.async_copy(obuf, acc.at[colv.at[b]], sem, add=True)

        def body(j, carry):
            pltpu.make_async_copy(obuf, acc.at[colv.at[j]], sem).wait()

            @pl.when(j + NBUF < NCH)
            def _():
                pltpu.async_copy(obuf, acc.at[colv.at[j + NBUF]], sem,
                                 add=True)
            return carry

        lax.fori_loop(0, NCH, body, 0)
        plsc.subcore_barrier()
        pltpu.sync_copy(acc.at[pl.ds(s * STRIPE, STRIPE)],
                        out.at[c, pl.ds(s * STRIPE, STRIPE)])

    return k


_prop_u = _make_prop(128, 2)
_prop_m = _make_prop(HP, 4)
_deg_kernel = _make_deg()


def _mm1_body(x_ref, w_ref, d0_ref, d1_ref, y0_ref, u_ref, sb_ref):
    y = jnp.dot(x_ref[...], w_ref[...], preferred_element_type=jnp.float32)
    deg = d0_ref[...][:, 0:1] + d1_ref[...][:, 0:1] + 1.0
    sb = jnp.broadcast_to(lax.rsqrt(deg), (MBLK, 128))
    sb_ref[...] = sb
    y0_ref[...] = y[:, 0:HP]
    u_ref[...] = y[:, HP:W3H] * sb


def _mm1(x, wt, d0, d1):
    fin = x.shape[1]
    return pl.pallas_call(
        _mm1_body,
        grid=(MT,),
        in_specs=[
            pl.BlockSpec((MBLK, fin), lambda i: (i, 0)),
            pl.BlockSpec((fin, W3H), lambda i: (0, 0)),
            pl.BlockSpec((MBLK, 16), lambda i: (i, 0)),
            pl.BlockSpec((MBLK, 16), lambda i: (i, 0)),
        ],
        out_specs=[
            pl.BlockSpec((MBLK, HP), lambda i: (i, 0)),
            pl.BlockSpec((MBLK, 128), lambda i: (i, 0)),
            pl.BlockSpec((MBLK, 128), lambda i: (i, 0)),
        ],
        out_shape=[
            jax.ShapeDtypeStruct((NP, HP), jnp.float32),
            jax.ShapeDtypeStruct((NP, 128), jnp.float32),
            jax.ShapeDtypeStruct((NP, 128), jnp.float32),
        ],
    )(x, wt, d0, d1)


def _mm2_body(g_ref, w_ref, sb_ref, y0_ref, u_ref):
    y = jnp.dot(g_ref[...], w_ref[...], preferred_element_type=jnp.float32)
    y0_ref[...] = y[:, 0:HP]
    u_ref[...] = y[:, HP:W3H] * sb_ref[...]


def _mm2(g, wt, sb):
    return pl.pallas_call(
        _mm2_body,
        grid=(MT,),
        in_specs=[
            pl.BlockSpec((MBLK, W3H), lambda i: (i, 0)),
            pl.BlockSpec((W3H, W3H), lambda i: (0, 0)),
            pl.BlockSpec((MBLK, 128), lambda i: (i, 0)),
        ],
        out_specs=[
            pl.BlockSpec((MBLK, HP), lambda i: (i, 0)),
            pl.BlockSpec((MBLK, 128), lambda i: (i, 0)),
        ],
        out_shape=[
            jax.ShapeDtypeStruct((NP, HP), jnp.float32),
            jax.ShapeDtypeStruct((NP, 128), jnp.float32),
        ],
    )(g, wt, sb)


def _mid_body(p_ref, u_ref, sb_ref, z1_ref, m_ref):
    u = u_ref[...]
    sb = sb_ref[...]
    ps = p_ref[0] + p_ref[1]
    z1_ref[...] = (ps[:, 0:HP] + u[:, 0:HP]) * sb[:, 0:HP]
    m_ref[...] = (ps[:, HP:128] + u[:, HP:128]) * (sb * sb)[:, HP:128]


def _mid(p, u, sb):
    return pl.pallas_call(
        _mid_body,
        grid=(MT,),
        in_specs=[
            pl.BlockSpec((NCORE, MBLK, 128), lambda i: (0, i, 0)),
            pl.BlockSpec((MBLK, 128), lambda i: (i, 0)),
            pl.BlockSpec((MBLK, 128), lambda i: (i, 0)),
        ],
        out_specs=[
            pl.BlockSpec((MBLK, HP), lambda i: (i, 0)),
            pl.BlockSpec((MBLK, HP), lambda i: (i, 0)),
        ],
        out_shape=[
            jax.ShapeDtypeStruct((NP, HP), jnp.float32),
            jax.ShapeDtypeStruct((NP, HP), jnp.float32),
        ],
    )(p, u, sb)


def _post_body(q0_ref, q1_ref, m_ref, y0_ref, z1_ref, sb_ref, gam_ref,
               dlt_ref, g_ref):
    z2 = (q0_ref[...] + q1_ref[...] + m_ref[...]) * sb_ref[...][:, 0:HP]
    z = jnp.concatenate([y0_ref[...], z1_ref[...], z2], axis=1)
    g_ref[...] = z * gam_ref[...][0:1, :] + dlt_ref[...][0:1, :]


def _post(q0, q1, m, y0, z1, sb, gam, dlt):
    return pl.pallas_call(
        _post_body,
        grid=(MT,),
        in_specs=[
            pl.BlockSpec((MBLK, HP), lambda i: (i, 0)),
            pl.BlockSpec((MBLK, HP), lambda i: (i, 0)),
            pl.BlockSpec((MBLK, HP), lambda i: (i, 0)),
            pl.BlockSpec((MBLK, HP), lambda i: (i, 0)),
            pl.BlockSpec((MBLK, HP), lambda i: (i, 0)),
            pl.BlockSpec((MBLK, 128), lambda i: (i, 0)),
            pl.BlockSpec((8, W3H), lambda i: (0, 0)),
            pl.BlockSpec((8, W3H), lambda i: (0, 0)),
        ],
        out_specs=pl.BlockSpec((MBLK, W3H), lambda i: (i, 0)),
        out_shape=jax.ShapeDtypeStruct((NP, W3H), jnp.float32),
    )(q0, q1, m, y0, z1, sb, gam, dlt)


def _final_body(g_ref, w_ref, b_ref, o_ref):
    o_ref[...] = (jnp.dot(g_ref[...], w_ref[...],
                          preferred_element_type=jnp.float32)
                  + b_ref[...][0:1, :])


def _final(g, lwt, lb):
    return pl.pallas_call(
        _final_body,
        grid=(MT,),
        in_specs=[
            pl.BlockSpec((MBLK, W3H), lambda i: (i, 0)),
            pl.BlockSpec((W3H, 128), lambda i: (0, 0)),
            pl.BlockSpec((8, 128), lambda i: (0, 0)),
        ],
        out_specs=pl.BlockSpec((MBLK, 128), lambda i: (i, 0)),
        out_shape=jax.ShapeDtypeStruct((NP, 128), jnp.float32),
    )(g, lwt, lb)


def _pack_w1(W1):
    w = jnp.pad(W1, ((0, 0), (0, HP - H), (0, 0)))
    return w.transpose(2, 0, 1).reshape(1433, W3H)


def _pack_w2(W2):
    w = W2.reshape(3, H, 3, H)
    w = jnp.pad(w, ((0, 0), (0, HP - H), (0, 0), (0, HP - H)))
    return w.transpose(2, 3, 0, 1).reshape(W3H, W3H)


def _pack_affine(b, bn_w, bn_b):
    g180 = bn_w / jnp.sqrt(1.0 + 1e-5)
    g3 = g180.reshape(3, H)
    d3 = b * g3 + bn_b.reshape(3, H)
    gam = jnp.pad(g3, ((0, 0), (0, HP - H))).reshape(W3H)
    dlt = jnp.pad(d3, ((0, 0), (0, HP - H))).reshape(W3H)
    return (jnp.broadcast_to(gam[None, :], (8, W3H)),
            jnp.broadcast_to(dlt[None, :], (8, W3H)))


def _layer(u_table, y0, sb, rowi, coli, gam, dlt):
    p = _prop_u(u_table, rowi, coli)
    z1, m = _mid(p, u_table, sb)
    q = _prop_m(m, rowi, coli)
    return _post(q[0], q[1], m, y0, z1, sb, gam, dlt)


def kernel(x, edge_index, W1, b1, bn1_w, bn1_b, W2, b2, bn2_w, bn2_b,
           W3, b3, bn3_w, bn3_b, lin_w, lin_b):
    rowp = jnp.concatenate(
        [edge_index[0], jnp.zeros((EP - E,), jnp.int32)]).reshape(NW, NCH, CH)
    colp = jnp.concatenate(
        [edge_index[1], jnp.full((EP - E,), N, jnp.int32)]).reshape(NW, NCH, CH)

    w1t = _pack_w1(W1)
    w2t = _pack_w2(W2)
    w3t = _pack_w2(W3)
    lr = jnp.pad(lin_w.reshape(7, 3, H), ((0, 0), (0, 0), (0, HP - H)))
    lwt = jnp.pad(lr.transpose(1, 2, 0).reshape(W3H, 7), ((0, 0), (0, 121)))
    lb = jnp.broadcast_to(jnp.pad(lin_b, (0, 121))[None, :], (8, 128))
    gam1, dlt1 = _pack_affine(b1, bn1_w, bn1_b)
    gam2, dlt2 = _pack_affine(b2, bn2_w, bn2_b)
    gam3, dlt3 = _pack_affine(b3, bn3_w, bn3_b)

    degp = _deg_kernel(colp)
    y0, u, sb = _mm1(x, w1t, degp[0], degp[1])
    g = _layer(u, y0, sb, rowp, colp, gam1, dlt1)
    y0, u = _mm2(g, w2t, sb)
    g = _layer(u, y0, sb, rowp, colp, gam2, dlt2)
    y0, u = _mm2(g, w3t, sb)
    g = _layer(u, y0, sb, rowp, colp, gam3, dlt3)
    out = _final(g, lwt, lb)
    return out[:N, :7]


# speedup vs baseline: 2.1314x; 2.1314x over previous
import functools

import jax
import jax.numpy as jnp
from jax import lax
from jax.experimental import pallas as pl
from jax.experimental.pallas import tpu as pltpu
from jax.experimental.pallas import tpu_sc as plsc

N = 10000
NP = 10240
E = 160000
NSUB = 16
NCORE = 2
NW = NCORE * NSUB
CH = 128
EP = 163840
NCH = EP // (NW * CH)
STRIPE = NP // NSUB
MBLK = 256
MT = NP // MBLK
NBUF = 4
F_PAD = 1536
H = 60
HP = 64
W3H = 3 * HP


def _fill_rows(ref, nrows, width, value):
    vec = jnp.full((16,), value, jnp.float32)

    def body(i, carry):
        for j in range(width // 16):
            ref[i, pl.ds(j * 16, 16)] = vec
        return carry

    lax.fori_loop(0, nrows, body, 0)


def _make_prop(width, nbuf):
    mesh = plsc.VectorSubcoreMesh(core_axis_name="c", subcore_axis_name="s")

    @functools.partial(
        pl.kernel,
        out_type=jax.ShapeDtypeStruct((NCORE, NP, width), jnp.float32),
        mesh=mesh,
        scratch_types=(
            [pltpu.VMEM((NCH, CH), jnp.int32)] * 2
            + [pltpu.VMEM((CH, width), jnp.float32)] * nbuf
            + [pltpu.VMEM_SHARED((NP, width), jnp.float32)]
            + [pltpu.SemaphoreType.DMA] * (nbuf + 1)
        ),
        compiler_params=pltpu.CompilerParams(use_tc_tiling_on_sc=False),
    )
    def k(table, rowi, coli, out, rowv, colv, *rest):
        gbufs = rest[:nbuf]
        acc = rest[nbuf]
        gsems = rest[nbuf + 1:2 * nbuf + 1]
        ssem = rest[2 * nbuf + 1]
        c = lax.axis_index("c")
        s = lax.axis_index("s")
        wid = c * NSUB + s
        _fill_rows(gbufs[0], CH, width, 0.0)
        for r in range(STRIPE // CH):
            pltpu.sync_copy(gbufs[0], acc.at[pl.ds(s * STRIPE + r * CH, CH)])
        pltpu.sync_copy(rowi.at[wid], rowv)
        pltpu.sync_copy(coli.at[wid], colv)
        plsc.subcore_barrier()

        for b in range(nbuf):
            pltpu.async_copy(table.at[rowv.at[b]], gbufs[b], gsems[b])

        def body(i, carry):
            for b in range(nbuf):
                j = nbuf * i + b
                pltpu.make_async_copy(
                    table.at[rowv.at[j]], gbufs[b], gsems[b]).wait()
                pltpu.async_copy(gbufs[b], acc.at[colv.at[j]], ssem, add=True)
                pltpu.make_async_copy(
                    gbufs[b], acc.at[colv.at[j]], ssem).wait()

                @pl.when(j + nbuf < NCH)
                def _():
                    pltpu.async_copy(table.at[rowv.at[j + nbuf]], gbufs[b],
                                     gsems[b])
            return carry

        lax.fori_loop(0, NCH // nbuf, body, 0)
        plsc.subcore_barrier()
        pltpu.sync_copy(acc.at[pl.ds(s * STRIPE, STRIPE)],
                        out.at[c, pl.ds(s * STRIPE, STRIPE)])

    return k


def _make_deg():
    width = 16
    mesh = plsc.VectorSubcoreMesh(core_axis_name="c", subcore_axis_name="s")

    @functools.partial(
        pl.kernel,
        out_type=jax.ShapeDtypeStruct((NCORE, NP, width), jnp.float32),
        mesh=mesh,
        scratch_types=[
            pltpu.VMEM((NCH, CH), jnp.int32),
            pltpu.VMEM((CH, width), jnp.float32),
            pltpu.VMEM((CH, width), jnp.float32),
            pltpu.VMEM_SHARED((NP, width), jnp.float32),
            pltpu.SemaphoreType.DMA,
        ],
        compiler_params=pltpu.CompilerParams(use_tc_tiling_on_sc=False),
    )
    def k(coli, out, colv, obuf, zbuf, acc, sem):
        c = lax.axis_index("c")
        s = lax.axis_index("s")
        wid = c * NSUB + s
        _fill_rows(zbuf, CH, width, 0.0)
        _fill_rows(obuf, CH, width, 1.0)
        for r in range(STRIPE // CH):
            pltpu.sync_copy(zbuf, acc.at[pl.ds(s * STRIPE + r * CH, CH)])
        pltpu.sync_copy(coli.at[wid], colv)
        plsc.subcore_barrier()

        for b in range(NBUF):
            pltpu.async_copy(obuf, acc.at[colv.at[b]], sem, add=True)

        def body(j, carry):
            pltpu.make_async_copy(obuf, acc.at[colv.at[j]], sem).wait()

            @pl.when(j + NBUF < NCH)
            def _():
                pltpu.async_copy(obuf, acc.at[colv.at[j + NBUF]], sem,
                                 add=True)
            return carry

        lax.fori_loop(0, NCH, body, 0)
        plsc.subcore_barrier()
        pltpu.sync_copy(acc.at[pl.ds(s * STRIPE, STRIPE)],
                        out.at[c, pl.ds(s * STRIPE, STRIPE)])

    return k


_prop_u = _make_prop(128, 2)
_prop_m = _make_prop(HP, 4)
_deg_kernel = _make_deg()


def _mm1_body(x_ref, w_ref, d0_ref, d1_ref, y0_ref, u_ref, sb_ref):
    y = jnp.dot(x_ref[...], w_ref[...], preferred_element_type=jnp.float32)
    deg = d0_ref[...][:, 0:1] + d1_ref[...][:, 0:1] + 1.0
    sb = jnp.broadcast_to(lax.rsqrt(deg), (MBLK, 128))
    sb_ref[...] = sb
    y0_ref[...] = y[:, 0:HP]
    u_ref[...] = y[:, HP:W3H] * sb


def _mm1(x, wt, d0, d1):
    fin = x.shape[1]
    return pl.pallas_call(
        _mm1_body,
        grid=(MT,),
        in_specs=[
            pl.BlockSpec((MBLK, fin), lambda i: (i, 0)),
            pl.BlockSpec((fin, W3H), lambda i: (0, 0)),
            pl.BlockSpec((MBLK, 16), lambda i: (i, 0)),
            pl.BlockSpec((MBLK, 16), lambda i: (i, 0)),
        ],
        out_specs=[
            pl.BlockSpec((MBLK, HP), lambda i: (i, 0)),
            pl.BlockSpec((MBLK, 128), lambda i: (i, 0)),
            pl.BlockSpec((MBLK, 128), lambda i: (i, 0)),
        ],
        out_shape=[
            jax.ShapeDtypeStruct((NP, HP), jnp.float32),
            jax.ShapeDtypeStruct((NP, 128), jnp.float32),
            jax.ShapeDtypeStruct((NP, 128), jnp.float32),
        ],
    )(x, wt, d0, d1)


def _mm2_body(g_ref, w_ref, sb_ref, y0_ref, u_ref):
    y = jnp.dot(g_ref[...], w_ref[...], preferred_element_type=jnp.float32)
    y0_ref[...] = y[:, 0:HP]
    u_ref[...] = y[:, HP:W3H] * sb_ref[...]


def _mm2(g, wt, sb):
    return pl.pallas_call(
        _mm2_body,
        grid=(MT,),
        in_specs=[
            pl.BlockSpec((MBLK, W3H), lambda i: (i, 0)),
            pl.BlockSpec((W3H, W3H), lambda i: (0, 0)),
            pl.BlockSpec((MBLK, 128), lambda i: (i, 0)),
        ],
        out_specs=[
            pl.BlockSpec((MBLK, HP), lambda i: (i, 0)),
            pl.BlockSpec((MBLK, 128), lambda i: (i, 0)),
        ],
        out_shape=[
            jax.ShapeDtypeStruct((NP, HP), jnp.float32),
            jax.ShapeDtypeStruct((NP, 128), jnp.float32),
        ],
    )(g, wt, sb)


def _mid_body(p_ref, u_ref, sb_ref, z1_ref, m_ref):
    u = u_ref[...]
    sb = sb_ref[...]
    ps = p_ref[0] + p_ref[1]
    z1_ref[...] = (ps[:, 0:HP] + u[:, 0:HP]) * sb[:, 0:HP]
    m_ref[...] = (ps[:, HP:128] + u[:, HP:128]) * (sb * sb)[:, HP:128]


def _mid(p, u, sb):
    return pl.pallas_call(
        _mid_body,
        grid=(MT,),
        in_specs=[
            pl.BlockSpec((NCORE, MBLK, 128), lambda i: (0, i, 0)),
            pl.BlockSpec((MBLK, 128), lambda i: (i, 0)),
            pl.BlockSpec((MBLK, 128), lambda i: (i, 0)),
        ],
        out_specs=[
            pl.BlockSpec((MBLK, HP), lambda i: (i, 0)),
            pl.BlockSpec((MBLK, HP), lambda i: (i, 0)),
        ],
        out_shape=[
            jax.ShapeDtypeStruct((NP, HP), jnp.float32),
            jax.ShapeDtypeStruct((NP, HP), jnp.float32),
        ],
    )(p, u, sb)


def _post_body(q0_ref, q1_ref, m_ref, y0_ref, z1_ref, sb_ref, gam_ref,
               dlt_ref, g_ref):
    z2 = (q0_ref[...] + q1_ref[...] + m_ref[...]) * sb_ref[...][:, 0:HP]
    z = jnp.concatenate([y0_ref[...], z1_ref[...], z2], axis=1)
    g_ref[...] = z * gam_ref[...][0:1, :] + dlt_ref[...][0:1, :]


def _post(q0, q1, m, y0, z1, sb, gam, dlt):
    return pl.pallas_call(
        _post_body,
        grid=(MT,),
        in_specs=[
            pl.BlockSpec((MBLK, HP), lambda i: (i, 0)),
            pl.BlockSpec((MBLK, HP), lambda i: (i, 0)),
            pl.BlockSpec((MBLK, HP), lambda i: (i, 0)),
            pl.BlockSpec((MBLK, HP), lambda i: (i, 0)),
            pl.BlockSpec((MBLK, HP), lambda i: (i, 0)),
            pl.BlockSpec((MBLK, 128), lambda i: (i, 0)),
            pl.BlockSpec((8, W3H), lambda i: (0, 0)),
            pl.BlockSpec((8, W3H), lambda i: (0, 0)),
        ],
        out_specs=pl.BlockSpec((MBLK, W3H), lambda i: (i, 0)),
        out_shape=jax.ShapeDtypeStruct((NP, W3H), jnp.float32),
    )(q0, q1, m, y0, z1, sb, gam, dlt)


def _final_body(g_ref, w_ref, b_ref, o_ref):
    o_ref[...] = (jnp.dot(g_ref[...], w_ref[...],
                          preferred_element_type=jnp.float32)
                  + b_ref[...][0:1, :])


def _final(g, lwt, lb):
    return pl.pallas_call(
        _final_body,
        grid=(MT,),
        in_specs=[
            pl.BlockSpec((MBLK, W3H), lambda i: (i, 0)),
            pl.BlockSpec((W3H, 128), lambda i: (0, 0)),
            pl.BlockSpec((8, 128), lambda i: (0, 0)),
        ],
        out_specs=pl.BlockSpec((MBLK, 128), lambda i: (i, 0)),
        out_shape=jax.ShapeDtypeStruct((NP, 128), jnp.float32),
    )(g, lwt, lb)


def _pack_w1(W1):
    w = jnp.pad(W1, ((0, 0), (0, HP - H), (0, 0)))
    return w.transpose(2, 0, 1).reshape(1433, W3H)


def _pack_w2(W2):
    w = W2.reshape(3, H, 3, H)
    w = jnp.pad(w, ((0, 0), (0, HP - H), (0, 0), (0, HP - H)))
    return w.transpose(2, 3, 0, 1).reshape(W3H, W3H)


def _pack_affine(b, bn_w, bn_b):
    g180 = bn_w / jnp.sqrt(1.0 + 1e-5)
    g3 = g180.reshape(3, H)
    d3 = b * g3 + bn_b.reshape(3, H)
    gam = jnp.pad(g3, ((0, 0), (0, HP - H))).reshape(W3H)
    dlt = jnp.pad(d3, ((0, 0), (0, HP - H))).reshape(W3H)
    return (jnp.broadcast_to(gam[None, :], (8, W3H)),
            jnp.broadcast_to(dlt[None, :], (8, W3H)))


def _layer(u_table, y0, sb, rowi, coli, gam, dlt):
    p = _prop_u(u_table, rowi, coli)
    z1, m = _mid(p, u_table, sb)
    q = _prop_m(m, rowi, coli)
    return _post(q[0], q[1], m, y0, z1, sb, gam, dlt)


def kernel(x, edge_index, W1, b1, bn1_w, bn1_b, W2, b2, bn2_w, bn2_b,
           W3, b3, bn3_w, bn3_b, lin_w, lin_b):
    pad = jnp.arange(EP - E, dtype=jnp.int32)
    rowp = jnp.concatenate(
        [edge_index[0], pad % N]).reshape(NW, NCH, CH)
    colp = jnp.concatenate(
        [edge_index[1], N + pad % (NP - N)]).reshape(NW, NCH, CH)

    w1t = _pack_w1(W1)
    w2t = _pack_w2(W2)
    w3t = _pack_w2(W3)
    lr = jnp.pad(lin_w.reshape(7, 3, H), ((0, 0), (0, 0), (0, HP - H)))
    lwt = jnp.pad(lr.transpose(1, 2, 0).reshape(W3H, 7), ((0, 0), (0, 121)))
    lb = jnp.broadcast_to(jnp.pad(lin_b, (0, 121))[None, :], (8, 128))
    gam1, dlt1 = _pack_affine(b1, bn1_w, bn1_b)
    gam2, dlt2 = _pack_affine(b2, bn2_w, bn2_b)
    gam3, dlt3 = _pack_affine(b3, bn3_w, bn3_b)

    degp = _deg_kernel(colp)
    y0, u, sb = _mm1(x, w1t, degp[0], degp[1])
    g = _layer(u, y0, sb, rowp, colp, gam1, dlt1)
    y0, u = _mm2(g, w2t, sb)
    g = _layer(u, y0, sb, rowp, colp, gam2, dlt2)
    y0, u = _mm2(g, w3t, sb)
    g = _layer(u, y0, sb, rowp, colp, gam3, dlt3)
    out = _final(g, lwt, lb)
    return out[:N, :7]


# speedup vs baseline: 2.3744x; 1.1140x over previous
import functools

import jax
import jax.numpy as jnp
from jax import lax
from jax.experimental import pallas as pl
from jax.experimental.pallas import tpu as pltpu
from jax.experimental.pallas import tpu_sc as plsc

N = 10000
NP = 10240
E = 160000
NSUB = 16
NCORE = 2
NW = NCORE * NSUB
CH = 128
EP = 163840
NCH = EP // (NW * CH)
STRIPE = NP // NSUB
MBLK = 256
MT = NP // MBLK
NBUF = 4
F_PAD = 1536
H = 60
HP = 64
W3H = 3 * HP


def _fill_rows(ref, nrows, width, value):
    vec = jnp.full((16,), value, jnp.float32)

    def body(i, carry):
        for j in range(width // 16):
            ref[i, pl.ds(j * 16, 16)] = vec
        return carry

    lax.fori_loop(0, nrows, body, 0)


def _make_prop(width, nbuf):
    mesh = plsc.VectorSubcoreMesh(core_axis_name="c", subcore_axis_name="s")

    @functools.partial(
        pl.kernel,
        out_type=jax.ShapeDtypeStruct((NCORE, NP, width), jnp.float32),
        mesh=mesh,
        scratch_types=(
            [pltpu.VMEM((NCH, CH), jnp.int32)] * 2
            + [pltpu.VMEM((CH, width), jnp.float32)] * nbuf
            + [pltpu.VMEM_SHARED((NP, width), jnp.float32)]
            + [pltpu.SemaphoreType.DMA] * (nbuf + 1)
        ),
        compiler_params=pltpu.CompilerParams(use_tc_tiling_on_sc=False),
    )
    def k(table, rowi, coli, out, rowv, colv, *rest):
        gbufs = rest[:nbuf]
        acc = rest[nbuf]
        gsems = rest[nbuf + 1:2 * nbuf + 1]
        ssem = rest[2 * nbuf + 1]
        c = lax.axis_index("c")
        s = lax.axis_index("s")
        wid = c * NSUB + s
        _fill_rows(gbufs[0], CH, width, 0.0)
        for r in range(STRIPE // CH):
            pltpu.sync_copy(gbufs[0], acc.at[pl.ds(s * STRIPE + r * CH, CH)])
        pltpu.sync_copy(rowi.at[wid], rowv)
        pltpu.sync_copy(coli.at[wid], colv)
        plsc.subcore_barrier()

        for b in range(nbuf):
            pltpu.async_copy(table.at[rowv.at[b]], gbufs[b], gsems[b])

        def body(i, carry):
            for b in range(nbuf):
                j = nbuf * i + b
                pltpu.make_async_copy(
                    table.at[rowv.at[j]], gbufs[b], gsems[b]).wait()
                pltpu.async_copy(gbufs[b], acc.at[colv.at[j]], ssem, add=True)
                pltpu.make_async_copy(
                    gbufs[b], acc.at[colv.at[j]], ssem).wait()

                @pl.when(j + nbuf < NCH)
                def _():
                    pltpu.async_copy(table.at[rowv.at[j + nbuf]], gbufs[b],
                                     gsems[b])
            return carry

        lax.fori_loop(0, NCH // nbuf, body, 0)
        plsc.subcore_barrier()
        pltpu.sync_copy(acc.at[pl.ds(s * STRIPE, STRIPE)],
                        out.at[c, pl.ds(s * STRIPE, STRIPE)])

    return k


def _make_deg():
    width = 16
    mesh = plsc.VectorSubcoreMesh(core_axis_name="c", subcore_axis_name="s")

    @functools.partial(
        pl.kernel,
        out_type=jax.ShapeDtypeStruct((NCORE, NP, width), jnp.float32),
        mesh=mesh,
        scratch_types=[
            pltpu.VMEM((NCH, CH), jnp.int32),
            pltpu.VMEM((CH, width), jnp.float32),
            pltpu.VMEM((CH, width), jnp.float32),
            pltpu.VMEM_SHARED((NP, width), jnp.float32),
            pltpu.SemaphoreType.DMA,
        ],
        compiler_params=pltpu.CompilerParams(use_tc_tiling_on_sc=False),
    )
    def k(coli, out, colv, obuf, zbuf, acc, sem):
        c = lax.axis_index("c")
        s = lax.axis_index("s")
        wid = c * NSUB + s
        _fill_rows(zbuf, CH, width, 0.0)
        _fill_rows(obuf, CH, width, 1.0)
        for r in range(STRIPE // CH):
            pltpu.sync_copy(zbuf, acc.at[pl.ds(s * STRIPE + r * CH, CH)])
        pltpu.sync_copy(coli.at[wid], colv)
        plsc.subcore_barrier()

        for b in range(NBUF):
            pltpu.async_copy(obuf, acc.at[colv.at[b]], sem, add=True)

        def body(j, carry):
            pltpu.make_async_copy(obuf, acc.at[colv.at[j]], sem).wait()

            @pl.when(j + NBUF < NCH)
            def _():
                pltpu.async_copy(obuf, acc.at[colv.at[j + NBUF]], sem,
                                 add=True)
            return carry

        lax.fori_loop(0, NCH, body, 0)
        plsc.subcore_barrier()
        pltpu.sync_copy(acc.at[pl.ds(s * STRIPE, STRIPE)],
                        out.at[c, pl.ds(s * STRIPE, STRIPE)])

    return k


_prop_u = _make_prop(128, 2)
_prop_m = _make_prop(HP, 4)
_deg_kernel = _make_deg()


def _mm1_body(x_ref, w_ref, d0_ref, d1_ref, y0_ref, u_ref, sb_ref):
    y = jnp.dot(x_ref[...], w_ref[...], preferred_element_type=jnp.float32)
    deg = d0_ref[...][:, 0:1] + d1_ref[...][:, 0:1] + 1.0
    sb = jnp.broadcast_to(lax.rsqrt(deg), (MBLK, 128))
    sb_ref[...] = sb
    y0_ref[...] = y[:, 0:HP]
    u_ref[...] = y[:, HP:W3H] * sb


def _mm1(x, wt, d0, d1):
    fin = x.shape[1]
    return pl.pallas_call(
        _mm1_body,
        grid=(MT,),
        in_specs=[
            pl.BlockSpec((MBLK, fin), lambda i: (i, 0)),
            pl.BlockSpec((fin, W3H), lambda i: (0, 0)),
            pl.BlockSpec((MBLK, 16), lambda i: (i, 0)),
            pl.BlockSpec((MBLK, 16), lambda i: (i, 0)),
        ],
        out_specs=[
            pl.BlockSpec((MBLK, HP), lambda i: (i, 0)),
            pl.BlockSpec((MBLK, 128), lambda i: (i, 0)),
            pl.BlockSpec((MBLK, 128), lambda i: (i, 0)),
        ],
        out_shape=[
            jax.ShapeDtypeStruct((NP, HP), jnp.float32),
            jax.ShapeDtypeStruct((NP, 128), jnp.float32),
            jax.ShapeDtypeStruct((NP, 128), jnp.float32),
        ],
    )(x, wt, d0, d1)


def _mm2_body(g_ref, w_ref, sb_ref, y0_ref, u_ref):
    y = jnp.dot(g_ref[...], w_ref[...], preferred_element_type=jnp.float32)
    y0_ref[...] = y[:, 0:HP]
    u_ref[...] = y[:, HP:W3H] * sb_ref[...]


def _mm2(g, wt, sb):
    return pl.pallas_call(
        _mm2_body,
        grid=(MT,),
        in_specs=[
            pl.BlockSpec((MBLK, W3H), lambda i: (i, 0)),
            pl.BlockSpec((W3H, W3H), lambda i: (0, 0)),
            pl.BlockSpec((MBLK, 128), lambda i: (i, 0)),
        ],
        out_specs=[
            pl.BlockSpec((MBLK, HP), lambda i: (i, 0)),
            pl.BlockSpec((MBLK, 128), lambda i: (i, 0)),
        ],
        out_shape=[
            jax.ShapeDtypeStruct((NP, HP), jnp.float32),
            jax.ShapeDtypeStruct((NP, 128), jnp.float32),
        ],
    )(g, wt, sb)


def _mid_body(p_ref, u_ref, sb_ref, z1_ref, m_ref):
    u = u_ref[...]
    sb = sb_ref[...]
    ps = p_ref[0] + p_ref[1]
    z1_ref[...] = (ps[:, 0:HP] + u[:, 0:HP]) * sb[:, 0:HP]
    m_ref[...] = (ps[:, HP:128] + u[:, HP:128]) * (sb * sb)[:, HP:128]


def _mid(p, u, sb):
    return pl.pallas_call(
        _mid_body,
        grid=(MT,),
        in_specs=[
            pl.BlockSpec((NCORE, MBLK, 128), lambda i: (0, i, 0)),
            pl.BlockSpec((MBLK, 128), lambda i: (i, 0)),
            pl.BlockSpec((MBLK, 128), lambda i: (i, 0)),
        ],
        out_specs=[
            pl.BlockSpec((MBLK, HP), lambda i: (i, 0)),
            pl.BlockSpec((MBLK, HP), lambda i: (i, 0)),
        ],
        out_shape=[
            jax.ShapeDtypeStruct((NP, HP), jnp.float32),
            jax.ShapeDtypeStruct((NP, HP), jnp.float32),
        ],
    )(p, u, sb)


def _postmm_body(q0_ref, q1_ref, m_ref, y0_ref, z1_ref, sb_ref, gam_ref,
                 dlt_ref, w_ref, y0o_ref, u_ref):
    z2 = (q0_ref[...] + q1_ref[...] + m_ref[...]) * sb_ref[...][:, 0:HP]
    z = jnp.concatenate([y0_ref[...], z1_ref[...], z2], axis=1)
    g = z * gam_ref[...][0:1, :] + dlt_ref[...][0:1, :]
    y = jnp.dot(g, w_ref[...], preferred_element_type=jnp.float32)
    y0o_ref[...] = y[:, 0:HP]
    u_ref[...] = y[:, HP:W3H] * sb_ref[...]


def _postmm(q0, q1, m, y0, z1, sb, gam, dlt, wt):
    return pl.pallas_call(
        _postmm_body,
        grid=(MT,),
        in_specs=[
            pl.BlockSpec((MBLK, HP), lambda i: (i, 0)),
            pl.BlockSpec((MBLK, HP), lambda i: (i, 0)),
            pl.BlockSpec((MBLK, HP), lambda i: (i, 0)),
            pl.BlockSpec((MBLK, HP), lambda i: (i, 0)),
            pl.BlockSpec((MBLK, HP), lambda i: (i, 0)),
            pl.BlockSpec((MBLK, 128), lambda i: (i, 0)),
            pl.BlockSpec((8, W3H), lambda i: (0, 0)),
            pl.BlockSpec((8, W3H), lambda i: (0, 0)),
            pl.BlockSpec((W3H, W3H), lambda i: (0, 0)),
        ],
        out_specs=[
            pl.BlockSpec((MBLK, HP), lambda i: (i, 0)),
            pl.BlockSpec((MBLK, 128), lambda i: (i, 0)),
        ],
        out_shape=[
            jax.ShapeDtypeStruct((NP, HP), jnp.float32),
            jax.ShapeDtypeStruct((NP, 128), jnp.float32),
        ],
    )(q0, q1, m, y0, z1, sb, gam, dlt, wt)


def _postfinal_body(q0_ref, q1_ref, m_ref, y0_ref, z1_ref, sb_ref, gam_ref,
                    dlt_ref, w_ref, b_ref, o_ref):
    z2 = (q0_ref[...] + q1_ref[...] + m_ref[...]) * sb_ref[...][:, 0:HP]
    z = jnp.concatenate([y0_ref[...], z1_ref[...], z2], axis=1)
    g = z * gam_ref[...][0:1, :] + dlt_ref[...][0:1, :]
    o_ref[...] = (jnp.dot(g, w_ref[...], preferred_element_type=jnp.float32)
                  + b_ref[...][0:1, :])


def _postfinal(q0, q1, m, y0, z1, sb, gam, dlt, lwt, lb):
    return pl.pallas_call(
        _postfinal_body,
        grid=(MT,),
        in_specs=[
            pl.BlockSpec((MBLK, HP), lambda i: (i, 0)),
            pl.BlockSpec((MBLK, HP), lambda i: (i, 0)),
            pl.BlockSpec((MBLK, HP), lambda i: (i, 0)),
            pl.BlockSpec((MBLK, HP), lambda i: (i, 0)),
            pl.BlockSpec((MBLK, HP), lambda i: (i, 0)),
            pl.BlockSpec((MBLK, 128), lambda i: (i, 0)),
            pl.BlockSpec((8, W3H), lambda i: (0, 0)),
            pl.BlockSpec((8, W3H), lambda i: (0, 0)),
            pl.BlockSpec((W3H, 128), lambda i: (0, 0)),
            pl.BlockSpec((8, 128), lambda i: (0, 0)),
        ],
        out_specs=pl.BlockSpec((MBLK, 128), lambda i: (i, 0)),
        out_shape=jax.ShapeDtypeStruct((NP, 128), jnp.float32),
    )(q0, q1, m, y0, z1, sb, gam, dlt, lwt, lb)


def _pack_w1(W1):
    w = jnp.pad(W1, ((0, 0), (0, HP - H), (0, 0)))
    return w.transpose(2, 0, 1).reshape(1433, W3H)


def _pack_w2(W2):
    w = W2.reshape(3, H, 3, H)
    w = jnp.pad(w, ((0, 0), (0, HP - H), (0, 0), (0, HP - H)))
    return w.transpose(2, 3, 0, 1).reshape(W3H, W3H)


def _pack_affine(b, bn_w, bn_b):
    g180 = bn_w / jnp.sqrt(1.0 + 1e-5)
    g3 = g180.reshape(3, H)
    d3 = b * g3 + bn_b.reshape(3, H)
    gam = jnp.pad(g3, ((0, 0), (0, HP - H))).reshape(W3H)
    dlt = jnp.pad(d3, ((0, 0), (0, HP - H))).reshape(W3H)
    return (jnp.broadcast_to(gam[None, :], (8, W3H)),
            jnp.broadcast_to(dlt[None, :], (8, W3H)))


def _hops(u_table, sb, rowi, coli):
    p = _prop_u(u_table, rowi, coli)
    z1, m = _mid(p, u_table, sb)
    q = _prop_m(m, rowi, coli)
    return z1, m, q


def kernel(x, edge_index, W1, b1, bn1_w, bn1_b, W2, b2, bn2_w, bn2_b,
           W3, b3, bn3_w, bn3_b, lin_w, lin_b):
    pad = jnp.arange(EP - E, dtype=jnp.int32)
    rowp = jnp.concatenate(
        [edge_index[0], pad % N]).reshape(NW, NCH, CH)
    colp = jnp.concatenate(
        [edge_index[1], N + pad % (NP - N)]).reshape(NW, NCH, CH)

    w1t = _pack_w1(W1)
    w2t = _pack_w2(W2)
    w3t = _pack_w2(W3)
    lr = jnp.pad(lin_w.reshape(7, 3, H), ((0, 0), (0, 0), (0, HP - H)))
    lwt = jnp.pad(lr.transpose(1, 2, 0).reshape(W3H, 7), ((0, 0), (0, 121)))
    lb = jnp.broadcast_to(jnp.pad(lin_b, (0, 121))[None, :], (8, 128))
    gam1, dlt1 = _pack_affine(b1, bn1_w, bn1_b)
    gam2, dlt2 = _pack_affine(b2, bn2_w, bn2_b)
    gam3, dlt3 = _pack_affine(b3, bn3_w, bn3_b)

    degp = _deg_kernel(colp)
    y0, u, sb = _mm1(x, w1t, degp[0], degp[1])
    z1, m, q = _hops(u, sb, rowp, colp)
    y0, u = _postmm(q[0], q[1], m, y0, z1, sb, gam1, dlt1, w2t)
    z1, m, q = _hops(u, sb, rowp, colp)
    y0, u = _postmm(q[0], q[1], m, y0, z1, sb, gam2, dlt2, w3t)
    z1, m, q = _hops(u, sb, rowp, colp)
    out = _postfinal(q[0], q[1], m, y0, z1, sb, gam3, dlt3, lwt, lb)
    return out[:N, :7]


# speedup vs baseline: 2.3748x; 1.0002x over previous
import functools

import jax
import jax.numpy as jnp
from jax import lax
from jax.experimental import pallas as pl
from jax.experimental.pallas import tpu as pltpu
from jax.experimental.pallas import tpu_sc as plsc

N = 10000
NP = 10240
E = 160000
NSUB = 16
NCORE = 2
NW = NCORE * NSUB
CH = 128
EP = 163840
NCH = EP // (NW * CH)
STRIPE = NP // NSUB
MBLK = 256
MT = NP // MBLK
NBUF = 4
F_PAD = 1536
H = 60
HP = 64
W3H = 3 * HP


def _fill_rows(ref, nrows, width, value):
    vec = jnp.full((16,), value, jnp.float32)

    def body(i, carry):
        for j in range(width // 16):
            ref[i, pl.ds(j * 16, 16)] = vec
        return carry

    lax.fori_loop(0, nrows, body, 0)


def _make_prop(width, nbuf):
    mesh = plsc.VectorSubcoreMesh(core_axis_name="c", subcore_axis_name="s")

    @functools.partial(
        pl.kernel,
        out_type=jax.ShapeDtypeStruct((NCORE, NP, width), jnp.float32),
        mesh=mesh,
        scratch_types=(
            [pltpu.VMEM((NCH, CH), jnp.int32)] * 2
            + [pltpu.VMEM((CH, width), jnp.float32)] * nbuf
            + [pltpu.VMEM_SHARED((NP, width), jnp.float32)]
            + [pltpu.SemaphoreType.DMA] * (nbuf + 1)
        ),
        compiler_params=pltpu.CompilerParams(use_tc_tiling_on_sc=False),
    )
    def k(table, rowi, coli, out, rowv, colv, *rest):
        gbufs = rest[:nbuf]
        acc = rest[nbuf]
        gsems = rest[nbuf + 1:2 * nbuf + 1]
        ssem = rest[2 * nbuf + 1]
        c = lax.axis_index("c")
        s = lax.axis_index("s")
        wid = c * NSUB + s
        _fill_rows(gbufs[0], CH, width, 0.0)
        for r in range(STRIPE // CH):
            pltpu.sync_copy(gbufs[0], acc.at[pl.ds(s * STRIPE + r * CH, CH)])
        pltpu.sync_copy(rowi.at[wid], rowv)
        pltpu.sync_copy(coli.at[wid], colv)
        plsc.subcore_barrier()

        for b in range(nbuf):
            pltpu.async_copy(table.at[rowv.at[b]], gbufs[b], gsems[b])

        def body(i, carry):
            for b in range(nbuf):
                j = nbuf * i + b
                pltpu.make_async_copy(
                    table.at[rowv.at[j]], gbufs[b], gsems[b]).wait()
                pltpu.async_copy(gbufs[b], acc.at[colv.at[j]], ssem, add=True)
                pltpu.make_async_copy(
                    gbufs[b], acc.at[colv.at[j]], ssem).wait()

                @pl.when(j + nbuf < NCH)
                def _():
                    pltpu.async_copy(table.at[rowv.at[j + nbuf]], gbufs[b],
                                     gsems[b])
            return carry

        lax.fori_loop(0, NCH // nbuf, body, 0)
        plsc.subcore_barrier()
        pltpu.sync_copy(acc.at[pl.ds(s * STRIPE, STRIPE)],
                        out.at[c, pl.ds(s * STRIPE, STRIPE)])

    return k


def _make_deg():
    width = 16
    mesh = plsc.VectorSubcoreMesh(core_axis_name="c", subcore_axis_name="s")

    @functools.partial(
        pl.kernel,
        out_type=jax.ShapeDtypeStruct((NCORE, NP, width), jnp.float32),
        mesh=mesh,
        scratch_types=[
            pltpu.VMEM((NCH, CH), jnp.int32),
            pltpu.VMEM((CH, width), jnp.float32),
            pltpu.VMEM((CH, width), jnp.float32),
            pltpu.VMEM_SHARED((NP, width), jnp.float32),
            pltpu.SemaphoreType.DMA,
        ],
        compiler_params=pltpu.CompilerParams(use_tc_tiling_on_sc=False),
    )
    def k(coli, out, colv, obuf, zbuf, acc, sem):
        c = lax.axis_index("c")
        s = lax.axis_index("s")
        wid = c * NSUB + s
        _fill_rows(zbuf, CH, width, 0.0)
        _fill_rows(obuf, CH, width, 1.0)
        for r in range(STRIPE // CH):
            pltpu.sync_copy(zbuf, acc.at[pl.ds(s * STRIPE + r * CH, CH)])
        pltpu.sync_copy(coli.at[wid], colv)
        plsc.subcore_barrier()

        for b in range(NBUF):
            pltpu.async_copy(obuf, acc.at[colv.at[b]], sem, add=True)

        def body(j, carry):
            pltpu.make_async_copy(obuf, acc.at[colv.at[j]], sem).wait()

            @pl.when(j + NBUF < NCH)
            def _():
                pltpu.async_copy(obuf, acc.at[colv.at[j + NBUF]], sem,
                                 add=True)
            return carry

        lax.fori_loop(0, NCH, body, 0)
        plsc.subcore_barrier()
        pltpu.sync_copy(acc.at[pl.ds(s * STRIPE, STRIPE)],
                        out.at[c, pl.ds(s * STRIPE, STRIPE)])

    return k


_prop_u = _make_prop(128, 2)
_prop_m = _make_prop(HP, 4)
_deg_kernel = _make_deg()


def _mm1_body(x_ref, w_ref, d0_ref, d1_ref, y0_ref, u_ref, sb_ref):
    y = jnp.dot(x_ref[...], w_ref[...], preferred_element_type=jnp.float32)
    deg = d0_ref[...][:, 0:1] + d1_ref[...][:, 0:1] + 1.0
    sb = jnp.broadcast_to(lax.rsqrt(deg), (MBLK, 128))
    sb_ref[...] = sb
    y0_ref[...] = y[:, 0:HP]
    u_ref[...] = y[:, HP:W3H] * sb


def _mm1(x, wt, d0, d1):
    fin = x.shape[1]
    return pl.pallas_call(
        _mm1_body,
        grid=(MT,),
        in_specs=[
            pl.BlockSpec((MBLK, fin), lambda i: (i, 0)),
            pl.BlockSpec((fin, W3H), lambda i: (0, 0)),
            pl.BlockSpec((MBLK, 16), lambda i: (i, 0)),
            pl.BlockSpec((MBLK, 16), lambda i: (i, 0)),
        ],
        out_specs=[
            pl.BlockSpec((MBLK, HP), lambda i: (i, 0)),
            pl.BlockSpec((MBLK, 128), lambda i: (i, 0)),
            pl.BlockSpec((MBLK, 128), lambda i: (i, 0)),
        ],
        out_shape=[
            jax.ShapeDtypeStruct((NP, HP), jnp.float32),
            jax.ShapeDtypeStruct((NP, 128), jnp.float32),
            jax.ShapeDtypeStruct((NP, 128), jnp.float32),
        ],
    )(x, wt, d0, d1)


def _mid_body(p_ref, u_ref, sb_ref, z1_ref, m_ref):
    u = u_ref[...]
    sb = sb_ref[...]
    ps = p_ref[0] + p_ref[1]
    z1_ref[...] = (ps[:, 0:HP] + u[:, 0:HP]) * sb[:, 0:HP]
    m_ref[...] = (ps[:, HP:128] + u[:, HP:128]) * (sb * sb)[:, HP:128]


def _mid(p, u, sb):
    return pl.pallas_call(
        _mid_body,
        grid=(MT,),
        in_specs=[
            pl.BlockSpec((NCORE, MBLK, 128), lambda i: (0, i, 0)),
            pl.BlockSpec((MBLK, 128), lambda i: (i, 0)),
            pl.BlockSpec((MBLK, 128), lambda i: (i, 0)),
        ],
        out_specs=[
            pl.BlockSpec((MBLK, HP), lambda i: (i, 0)),
            pl.BlockSpec((MBLK, HP), lambda i: (i, 0)),
        ],
        out_shape=[
            jax.ShapeDtypeStruct((NP, HP), jnp.float32),
            jax.ShapeDtypeStruct((NP, HP), jnp.float32),
        ],
    )(p, u, sb)


def _postmm_body(q0_ref, q1_ref, m_ref, y0_ref, z1_ref, sb_ref, gam_ref,
                 dlt_ref, w_ref, y0o_ref, u_ref):
    z2 = (q0_ref[...] + q1_ref[...] + m_ref[...]) * sb_ref[...][:, 0:HP]
    z = jnp.concatenate([y0_ref[...], z1_ref[...], z2], axis=1)
    g = z * gam_ref[...][0:1, :] + dlt_ref[...][0:1, :]
    y = jnp.dot(g, w_ref[...], preferred_element_type=jnp.float32)
    y0o_ref[...] = y[:, 0:HP]
    u_ref[...] = y[:, HP:W3H] * sb_ref[...]


def _postmm(q0, q1, m, y0, z1, sb, gam, dlt, wt):
    return pl.pallas_call(
        _postmm_body,
        grid=(MT,),
        in_specs=[
            pl.BlockSpec((MBLK, HP), lambda i: (i, 0)),
            pl.BlockSpec((MBLK, HP), lambda i: (i, 0)),
            pl.BlockSpec((MBLK, HP), lambda i: (i, 0)),
            pl.BlockSpec((MBLK, HP), lambda i: (i, 0)),
            pl.BlockSpec((MBLK, HP), lambda i: (i, 0)),
            pl.BlockSpec((MBLK, 128), lambda i: (i, 0)),
            pl.BlockSpec((8, W3H), lambda i: (0, 0)),
            pl.BlockSpec((8, W3H), lambda i: (0, 0)),
            pl.BlockSpec((W3H, W3H), lambda i: (0, 0)),
        ],
        out_specs=[
            pl.BlockSpec((MBLK, HP), lambda i: (i, 0)),
            pl.BlockSpec((MBLK, 128), lambda i: (i, 0)),
        ],
        out_shape=[
            jax.ShapeDtypeStruct((NP, HP), jnp.float32),
            jax.ShapeDtypeStruct((NP, 128), jnp.float32),
        ],
    )(q0, q1, m, y0, z1, sb, gam, dlt, wt)


def _postfinal_body(q0_ref, q1_ref, m_ref, y0_ref, z1_ref, sb_ref, gam_ref,
                    dlt_ref, w_ref, b_ref, o_ref):
    z2 = (q0_ref[...] + q1_ref[...] + m_ref[...]) * sb_ref[...][:, 0:HP]
    z = jnp.concatenate([y0_ref[...], z1_ref[...], z2], axis=1)
    g = z * gam_ref[...][0:1, :] + dlt_ref[...][0:1, :]
    o_ref[...] = (jnp.dot(g, w_ref[...], preferred_element_type=jnp.float32)
                  + b_ref[...][0:1, :])


def _postfinal(q0, q1, m, y0, z1, sb, gam, dlt, lwt, lb):
    return pl.pallas_call(
        _postfinal_body,
        grid=(MT,),
        in_specs=[
            pl.BlockSpec((MBLK, HP), lambda i: (i, 0)),
            pl.BlockSpec((MBLK, HP), lambda i: (i, 0)),
            pl.BlockSpec((MBLK, HP), lambda i: (i, 0)),
            pl.BlockSpec((MBLK, HP), lambda i: (i, 0)),
            pl.BlockSpec((MBLK, HP), lambda i: (i, 0)),
            pl.BlockSpec((MBLK, 128), lambda i: (i, 0)),
            pl.BlockSpec((8, W3H), lambda i: (0, 0)),
            pl.BlockSpec((8, W3H), lambda i: (0, 0)),
            pl.BlockSpec((W3H, 128), lambda i: (0, 0)),
            pl.BlockSpec((8, 128), lambda i: (0, 0)),
        ],
        out_specs=pl.BlockSpec((MBLK, 128), lambda i: (i, 0)),
        out_shape=jax.ShapeDtypeStruct((NP, 128), jnp.float32),
    )(q0, q1, m, y0, z1, sb, gam, dlt, lwt, lb)


def _pack_w1(W1):
    w = jnp.pad(W1, ((0, 0), (0, HP - H), (0, 0)))
    return w.transpose(2, 0, 1).reshape(1433, W3H)


def _pack_w2(W2):
    w = W2.reshape(3, H, 3, H)
    w = jnp.pad(w, ((0, 0), (0, HP - H), (0, 0), (0, HP - H)))
    return w.transpose(2, 3, 0, 1).reshape(W3H, W3H)


def _pack_affine(b, bn_w, bn_b):
    g180 = bn_w / jnp.sqrt(1.0 + 1e-5)
    g3 = g180.reshape(3, H)
    d3 = b * g3 + bn_b.reshape(3, H)
    gam = jnp.pad(g3, ((0, 0), (0, HP - H))).reshape(W3H)
    dlt = jnp.pad(d3, ((0, 0), (0, HP - H))).reshape(W3H)
    return (jnp.broadcast_to(gam[None, :], (8, W3H)),
            jnp.broadcast_to(dlt[None, :], (8, W3H)))


def _hops(u_table, sb, rowi, coli):
    p = _prop_u(u_table, rowi, coli)
    z1, m = _mid(p, u_table, sb)
    q = _prop_m(m, rowi, coli)
    return z1, m, q


def kernel(x, edge_index, W1, b1, bn1_w, bn1_b, W2, b2, bn2_w, bn2_b,
           W3, b3, bn3_w, bn3_b, lin_w, lin_b):
    pad = jnp.arange(EP - E, dtype=jnp.int32)
    rowp = jnp.concatenate(
        [edge_index[0], pad % N]).reshape(NW, NCH, CH)
    colp = jnp.concatenate(
        [edge_index[1], N + pad % (NP - N)]).reshape(NW, NCH, CH)

    w1t = _pack_w1(W1)
    w2t = _pack_w2(W2)
    w3t = _pack_w2(W3)
    lr = jnp.pad(lin_w.reshape(7, 3, H), ((0, 0), (0, 0), (0, HP - H)))
    lwt = jnp.pad(lr.transpose(1, 2, 0).reshape(W3H, 7), ((0, 0), (0, 121)))
    lb = jnp.broadcast_to(jnp.pad(lin_b, (0, 121))[None, :], (8, 128))
    gam1, dlt1 = _pack_affine(b1, bn1_w, bn1_b)
    gam2, dlt2 = _pack_affine(b2, bn2_w, bn2_b)
    gam3, dlt3 = _pack_affine(b3, bn3_w, bn3_b)

    degp = _deg_kernel(colp)
    y0, u, sb = _mm1(x, w1t, degp[0], degp[1])
    z1, m, q = _hops(u, sb, rowp, colp)
    y0, u = _postmm(q[0], q[1], m, y0, z1, sb, gam1, dlt1, w2t)
    z1, m, q = _hops(u, sb, rowp, colp)
    y0, u = _postmm(q[0], q[1], m, y0, z1, sb, gam2, dlt2, w3t)
    z1, m, q = _hops(u, sb, rowp, colp)
    out = _postfinal(q[0], q[1], m, y0, z1, sb, gam3, dlt3, lwt, lb)
    return out[:N, :7]
